# baseline (speedup 1.0000x reference)
import functools

import jax
import jax.numpy as jnp
from jax import lax
from jax.experimental import pallas as pl
from jax.experimental.pallas import tpu as pltpu
from jax.experimental.pallas import tpu_sc as plsc

_NUM_CORES = 2
_NUM_SUBCORES = 16
_NW = _NUM_CORES * _NUM_SUBCORES


def _embedding_lookup(subject_ids, embedding_weight):
    batch, = subject_ids.shape
    _, embed_dim = embedding_weight.shape
    b_per_w = batch // _NW

    mesh = plsc.VectorSubcoreMesh(core_axis_name="c", subcore_axis_name="s")

    @functools.partial(
        pl.kernel,
        mesh=mesh,
        out_type=jax.ShapeDtypeStruct((batch, embed_dim), jnp.float32),
        scratch_types=[
            pltpu.VMEM((b_per_w,), jnp.int32),
            pltpu.VMEM((b_per_w, embed_dim), jnp.float32),
            pltpu.SemaphoreType.DMA,
        ],
        compiler_params=pltpu.CompilerParams(use_tc_tiling_on_sc=False),
    )
    def lookup(ids_hbm, table_hbm, out_hbm, idx_v, rows_v, sem):
        wid = lax.axis_index("s") * _NUM_CORES + lax.axis_index("c")
        base = wid * b_per_w
        pltpu.sync_copy(ids_hbm.at[pl.ds(base, b_per_w)], idx_v)
        pltpu.async_copy(table_hbm.at[idx_v], rows_v, sem).wait()
        pltpu.sync_copy(rows_v, out_hbm.at[pl.ds(base, b_per_w)])

    return lookup(subject_ids, embedding_weight)


def kernel(subject_ids, embedding_weight):
    return _embedding_lookup(subject_ids.astype(jnp.int32), embedding_weight)


# speedup vs baseline: 1.4718x; 1.4718x over previous
import functools

import jax
import jax.numpy as jnp
from jax import lax
from jax.experimental import pallas as pl
from jax.experimental.pallas import tpu as pltpu
from jax.experimental.pallas import tpu_sc as plsc

_NUM_CORES = 2
_NUM_SUBCORES = 16
_NW = _NUM_CORES * _NUM_SUBCORES
_LANES = 16


def _embedding_lookup(subject_ids, embedding_weight):
    batch, = subject_ids.shape
    _, embed_dim = embedding_weight.shape
    b_per_w = batch // _NW
    groups = b_per_w // _LANES

    mesh = plsc.VectorSubcoreMesh(core_axis_name="c", subcore_axis_name="s")

    @functools.partial(
        pl.kernel,
        mesh=mesh,
        out_type=jax.ShapeDtypeStruct((batch, embed_dim), jnp.float32),
        scratch_types=[
            pltpu.VMEM((b_per_w,), jnp.int32),
            pltpu.VMEM((b_per_w, embed_dim), jnp.float32),
            pltpu.SemaphoreType.DMA,
        ],
    )
    def lookup(ids_hbm, table_hbm, out_hbm, idx_v, rows_v, sem):
        wid = lax.axis_index("s") * _NUM_CORES + lax.axis_index("c")
        base = wid * b_per_w
        pltpu.sync_copy(ids_hbm.at[pl.ds(base, b_per_w)], idx_v)

        def body(g, carry):
            vec = idx_v[pl.ds(g * _LANES, _LANES)]
            for lane in range(_LANES):
                r = vec[lane]
                pltpu.async_copy(
                    table_hbm.at[pl.ds(r, 1)],
                    rows_v.at[pl.ds(g * _LANES + lane, 1)],
                    sem,
                )
            return carry

        lax.fori_loop(0, groups, body, 0)
        pltpu.make_async_copy(
            table_hbm.at[pl.ds(0, b_per_w)], rows_v, sem
        ).wait()
        pltpu.sync_copy(rows_v, out_hbm.at[pl.ds(base, b_per_w)])

    return lookup(subject_ids, embedding_weight)


def kernel(subject_ids, embedding_weight):
    return _embedding_lookup(subject_ids.astype(jnp.int32), embedding_weight)


# speedup vs baseline: 1.9501x; 1.3250x over previous
import functools

import jax
import jax.numpy as jnp
from jax import lax
from jax.experimental import pallas as pl
from jax.experimental.pallas import tpu as pltpu
from jax.experimental.pallas import tpu_sc as plsc

_NUM_CORES = 2
_NUM_SUBCORES = 16
_NW = _NUM_CORES * _NUM_SUBCORES
_LANES = 16
_CHUNK = 8192


def _embedding_lookup_t(subject_ids, table_t):
    embed_dim, num_rows = table_t.shape
    batch, = subject_ids.shape
    feats_per_w = embed_dim // _NW
    n_chunks = batch // _CHUNK
    groups = _CHUNK // _LANES

    mesh = plsc.VectorSubcoreMesh(core_axis_name="c", subcore_axis_name="s")

    @functools.partial(
        pl.kernel,
        mesh=mesh,
        out_type=jax.ShapeDtypeStruct((embed_dim, batch), jnp.float32),
        scratch_types=[
            pltpu.VMEM((num_rows,), jnp.float32),
            pltpu.VMEM((_CHUNK,), jnp.int32),
            pltpu.VMEM((_CHUNK,), jnp.float32),
        ],
        compiler_params=pltpu.CompilerParams(needs_layout_passes=False),
    )
    def lookup(ids_hbm, table_hbm, out_hbm, row_v, idx_v, val_v):
        wid = lax.axis_index("s") * _NUM_CORES + lax.axis_index("c")
        for k in range(feats_per_w):
            j = wid * feats_per_w + k
            pltpu.sync_copy(table_hbm.at[j], row_v)
            for c in range(n_chunks):
                pltpu.sync_copy(ids_hbm.at[pl.ds(c * _CHUNK, _CHUNK)], idx_v)

                def body(g, carry):
                    vec = idx_v[pl.ds(g * _LANES, _LANES)]
                    val_v[pl.ds(g * _LANES, _LANES)] = plsc.load_gather(
                        row_v, [vec]
                    )
                    return carry

                lax.fori_loop(0, groups, body, 0)
                pltpu.sync_copy(
                    val_v, out_hbm.at[j, pl.ds(c * _CHUNK, _CHUNK)]
                )

    return lookup(subject_ids, table_t)


def kernel(subject_ids, embedding_weight):
    out_t = _embedding_lookup_t(
        subject_ids.astype(jnp.int32), embedding_weight.T
    )
    return out_t.T


# speedup vs baseline: 2.6869x; 1.3779x over previous
import functools

import jax
import jax.numpy as jnp
from jax import lax
from jax.experimental import pallas as pl
from jax.experimental.pallas import tpu as pltpu
from jax.experimental.pallas import tpu_sc as plsc

_NUM_CORES = 2
_NUM_SUBCORES = 16
_NW = _NUM_CORES * _NUM_SUBCORES
_LANES = 16
_CHUNK = 4096


def _embedding_lookup_t(subject_ids, table_t):
    embed_dim, num_rows = table_t.shape
    batch, = subject_ids.shape
    feats_per_w = embed_dim // _NW
    n_chunks = batch // _CHUNK
    groups = _CHUNK // _LANES

    mesh = plsc.VectorSubcoreMesh(core_axis_name="c", subcore_axis_name="s")

    @functools.partial(
        pl.kernel,
        mesh=mesh,
        out_type=jax.ShapeDtypeStruct((embed_dim, batch), jnp.float32),
        scratch_types=[
            pltpu.VMEM((num_rows,), jnp.float32),
            pltpu.VMEM((batch,), jnp.int32),
            pltpu.VMEM((2, _CHUNK), jnp.float32),
            pltpu.SemaphoreType.DMA,
            pltpu.SemaphoreType.DMA,
            pltpu.SemaphoreType.DMA,
        ],
        compiler_params=pltpu.CompilerParams(needs_layout_passes=False),
    )
    def lookup(ids_hbm, table_hbm, out_hbm, row_v, idx_v, val_v,
               sem_i, sem_r, sem_w):
        wid = lax.axis_index("s") * _NUM_CORES + lax.axis_index("c")
        ids_cp = pltpu.make_async_copy(ids_hbm, idx_v, sem_i)
        ids_cp.start()
        row_cp0 = pltpu.make_async_copy(
            table_hbm.at[wid * feats_per_w], row_v, sem_r
        )
        row_cp0.start()
        ids_cp.wait()
        pending = []
        for k in range(feats_per_w):
            j = wid * feats_per_w + k
            if k == 0:
                row_cp0.wait()
            else:
                pltpu.make_async_copy(table_hbm.at[j], row_v, sem_r).wait()
            for c in range(n_chunks):
                buf = c % 2
                if len(pending) >= 2:
                    pending.pop(0).wait()

                @plsc.parallel_loop(0, groups, unroll=8)
                def _gather(g, _c=c, _buf=buf):
                    vec = idx_v[pl.ds(_c * _CHUNK + g * _LANES, _LANES)]
                    val_v[_buf, pl.ds(g * _LANES, _LANES)] = plsc.load_gather(
                        row_v, [vec]
                    )

                last_row = k == feats_per_w - 1 and c == n_chunks - 1
                if c == n_chunks - 1 and not last_row:
                    pltpu.make_async_copy(
                        table_hbm.at[j + 1], row_v, sem_r
                    ).start()
                w = pltpu.make_async_copy(
                    val_v.at[buf],
                    out_hbm.at[j, pl.ds(c * _CHUNK, _CHUNK)],
                    sem_w,
                )
                w.start()
                pending.append(w)
        while pending:
            pending.pop(0).wait()

    return lookup(subject_ids, table_t)


def kernel(subject_ids, embedding_weight):
    out_t = _embedding_lookup_t(
        subject_ids.astype(jnp.int32), embedding_weight.T
    )
    return out_t.T
